# Initial kernel scaffold; baseline (speedup 1.0000x reference)
import jax
import jax.numpy as jnp
from jax.experimental import pallas as pl


def kernel(x, edge_index, W1, b1, W2, b2, M1, mb1, M2, mb2, M3, mb3):
    raise NotImplementedError("write your pallas kernel here")


# speedup vs baseline: 10.3596x; 10.3596x over previous
import functools

import jax
import jax.numpy as jnp
from jax import lax
from jax.experimental import pallas as pl
from jax.experimental.pallas import tpu as pltpu
from jax.experimental.pallas import tpu_sc as plsc

N = 10000
NP = 10240
E = 320000
D = 128
NC = 2
NS = 16
NW = NC * NS
EPW = E // NW
CH = 80
NCHUNK = EPW // CH
RPT = NP // NS

_MESH = plsc.VectorSubcoreMesh(
    core_axis_name="c", subcore_axis_name="s", num_cores=NC, num_subcores=NS)


def _wid():
  return lax.axis_index("c") * NS + lax.axis_index("s")


@functools.partial(
    pl.kernel,
    out_type=jax.ShapeDtypeStruct((NC, NP, 16), jnp.float32),
    mesh=_MESH,
    scratch_types=[
        pltpu.VMEM((CH,), jnp.int32),
        pltpu.VMEM((CH, 16), jnp.float32),
        pltpu.VMEM((128, 16), jnp.float32),
        pltpu.VMEM_SHARED((NP, 16), jnp.float32),
    ],
)
def _deg_kernel(dst_hbm, out_hbm, idx_v, ones_v, zero_v, acc_sh):
  c = lax.axis_index("c")
  s = lax.axis_index("s")
  wid = c * NS + s
  one = jnp.ones((16,), jnp.float32)
  zero = jnp.zeros((16,), jnp.float32)
  for i in range(CH):
    ones_v[i] = one
  for i in range(128):
    zero_v[i] = zero
  for k in range(RPT // 128):
    pltpu.sync_copy(zero_v, acc_sh.at[pl.ds(s * RPT + k * 128, 128)])
  plsc.subcore_barrier()

  def body(k, carry):
    base = pl.multiple_of(wid * EPW + k * CH, 8)
    pltpu.sync_copy(dst_hbm.at[pl.ds(base, CH)], idx_v)
    pltpu.sync_copy(ones_v, acc_sh.at[idx_v], add=True)
    return carry

  lax.fori_loop(0, NCHUNK, body, 0)
  plsc.subcore_barrier()
  pltpu.sync_copy(acc_sh.at[pl.ds(s * RPT, RPT)],
                  out_hbm.at[c].at[pl.ds(s * RPT, RPT)])


@functools.partial(
    pl.kernel,
    out_type=jax.ShapeDtypeStruct((NC, NP, D), jnp.float32),
    mesh=_MESH,
    scratch_types=[
        pltpu.VMEM((CH,), jnp.int32),
        pltpu.VMEM((CH,), jnp.int32),
        pltpu.VMEM((CH, D), jnp.float32),
        pltpu.VMEM((32, D), jnp.float32),
        pltpu.VMEM_SHARED((NP, D), jnp.float32),
        pltpu.SemaphoreType.DMA,
    ],
)
def _agg_kernel(y_hbm, src_hbm, dst_hbm, out_hbm,
                src_v, dst_v, rows_v, zero_v, acc_sh, sem):
  c = lax.axis_index("c")
  s = lax.axis_index("s")
  wid = c * NS + s
  zero = jnp.zeros((16,), jnp.float32)
  for i in range(32):
    for j in range(D // 16):
      zero_v[i, pl.ds(j * 16, 16)] = zero
  for k in range(RPT // 32):
    pltpu.sync_copy(zero_v, acc_sh.at[pl.ds(s * RPT + k * 32, 32)])
  plsc.subcore_barrier()

  def body(k, carry):
    base = pl.multiple_of(wid * EPW + k * CH, 8)
    pltpu.sync_copy(src_hbm.at[pl.ds(base, CH)], src_v)
    pltpu.sync_copy(dst_hbm.at[pl.ds(base, CH)], dst_v)
    pltpu.async_copy(y_hbm.at[src_v], rows_v, sem).wait()
    pltpu.sync_copy(rows_v, acc_sh.at[dst_v], add=True)
    return carry

  lax.fori_loop(0, NCHUNK, body, 0)
  plsc.subcore_barrier()
  pltpu.sync_copy(acc_sh.at[pl.ds(s * RPT, RPT)],
                  out_hbm.at[c].at[pl.ds(s * RPT, RPT)])


_EC = 2000


@functools.partial(
    pl.kernel,
    out_type=jax.ShapeDtypeStruct((E * 4,), jnp.float32),
    mesh=_MESH,
    scratch_types=[
        pltpu.VMEM((N * 8,), jnp.float32),
        pltpu.VMEM((_EC,), jnp.int32),
        pltpu.VMEM((_EC,), jnp.int32),
        pltpu.VMEM((_EC * 4,), jnp.float32),
    ],
    compiler_params=pltpu.CompilerParams(needs_layout_passes=False),
)
def _edge_out_kernel(ab_hbm, src_hbm, dst_hbm, out_hbm,
                     ab_v, src_v, dst_v, out_v):
  wid = _wid()
  pltpu.sync_copy(ab_hbm, ab_v)
  lanes = lax.iota(jnp.int32, 16)

  def chunk(kc, carry):
    base = pl.multiple_of(wid * EPW + kc * _EC, 8)
    pltpu.sync_copy(src_hbm.at[pl.ds(base, _EC)], src_v)
    pltpu.sync_copy(dst_hbm.at[pl.ds(base, _EC)], dst_v)

    def group(g, carry2):
      off = pl.multiple_of(g * 16, 16)
      sv = src_v[pl.ds(off, 16)] * 8
      dv = dst_v[pl.ds(off, 16)] * 8
      erow = (g * 16 + lanes) * 4
      for j in range(4):
        a = plsc.load_gather(ab_v, [sv + j])
        b = plsc.load_gather(ab_v, [dv + (j + 4)])
        plsc.store_scatter(out_v, [erow + j], a + b)
      return carry2

    lax.fori_loop(0, _EC // 16, group, 0)
    pltpu.sync_copy(out_v, out_hbm.at[pl.ds(base * 4, _EC * 4)])
    return carry

  lax.fori_loop(0, EPW // _EC, chunk, 0)


_BLK = 1000
_GRID = N // _BLK


def _dinv_block(degp):
  deg = degp[0, :, 0] + degp[1, :, 0] + 1.0
  return lax.rsqrt(deg)[:, None]


def _tc1_body(x_ref, w_ref, degp_ref, y_ref):
  dinv = _dinv_block(degp_ref[...])
  y_ref[...] = jnp.dot(x_ref[...], w_ref[...],
                       preferred_element_type=jnp.float32) * dinv


def _tc2_body(accp_ref, y_ref, degp_ref, b_ref, w_ref, y2_ref):
  dinv = _dinv_block(degp_ref[...])
  a = accp_ref[...]
  h = jnp.maximum(dinv * (a[0] + a[1] + y_ref[...]) + b_ref[...], 0.0)
  y2_ref[...] = jnp.dot(h, w_ref[...],
                        preferred_element_type=jnp.float32) * dinv


def _tc3_body(accp_ref, y_ref, degp_ref, b_ref,
              m1_ref, m2_ref, m3_ref, mb1_ref, mb2_ref, mb3_ref, ab_ref):
  dinv = _dinv_block(degp_ref[...])
  a = accp_ref[...]
  h2 = jnp.maximum(dinv * (a[0] + a[1] + y_ref[...]) + b_ref[...], 0.0)
  m23 = jnp.dot(m2_ref[...], m3_ref[...], preferred_element_type=jnp.float32)
  mc = jnp.dot(m1_ref[...], m23, preferred_element_type=jnp.float32)
  bc = (jnp.dot(mb1_ref[...], m23, preferred_element_type=jnp.float32)
        + jnp.dot(mb2_ref[...], m3_ref[...],
                  preferred_element_type=jnp.float32)
        + mb3_ref[...])
  mcc = jnp.concatenate([mc[:D], mc[D:]], axis=1)
  ab = jnp.dot(h2, mcc, preferred_element_type=jnp.float32)
  ab_ref[...] = ab + jnp.concatenate(
      [jnp.zeros((1, 4), jnp.float32), bc], axis=1)


def _row_spec(w=D):
  return pl.BlockSpec((_BLK, w), lambda i: (i, 0))


_DEGP_SPEC = pl.BlockSpec((NC, _BLK, 16), lambda i: (0, i, 0))
_ACCP_SPEC = pl.BlockSpec((NC, _BLK, D), lambda i: (0, i, 0))


def _full(*shape):
  return pl.BlockSpec(shape, lambda i: tuple(0 for _ in shape))


def kernel(x, edge_index, W1, b1, W2, b2, M1, mb1, M2, mb2, M3, mb3):
  src = edge_index[0].astype(jnp.int32)
  dst = edge_index[1].astype(jnp.int32)

  degp = _deg_kernel(dst)

  y1 = pl.pallas_call(
      _tc1_body,
      grid=(_GRID,),
      in_specs=[_row_spec(), _full(D, D), _DEGP_SPEC],
      out_specs=_row_spec(),
      out_shape=jax.ShapeDtypeStruct((N, D), jnp.float32),
  )(x, W1, degp)

  acc1 = _agg_kernel(y1, src, dst)

  y2 = pl.pallas_call(
      _tc2_body,
      grid=(_GRID,),
      in_specs=[_ACCP_SPEC, _row_spec(), _DEGP_SPEC, _full(1, D),
                _full(D, D)],
      out_specs=_row_spec(),
      out_shape=jax.ShapeDtypeStruct((N, D), jnp.float32),
  )(acc1, y1, degp, b1.reshape(1, D), W2)

  acc2 = _agg_kernel(y2, src, dst)

  ab = pl.pallas_call(
      _tc3_body,
      grid=(_GRID,),
      in_specs=[_ACCP_SPEC, _row_spec(), _DEGP_SPEC, _full(1, D),
                _full(2 * D, 64), _full(64, 32), _full(32, 4),
                _full(1, 64), _full(1, 32), _full(1, 4)],
      out_specs=_row_spec(8),
      out_shape=jax.ShapeDtypeStruct((N, 8), jnp.float32),
  )(acc2, y2, degp, b2.reshape(1, D), M1, M2, M3,
    mb1.reshape(1, 64), mb2.reshape(1, 32), mb3.reshape(1, 4))

  return _edge_out_kernel(ab.reshape(N * 8), src, dst).reshape(E, 4)


# speedup vs baseline: 17.5851x; 1.6975x over previous
import functools

import jax
import jax.numpy as jnp
from jax import lax
from jax.experimental import pallas as pl
from jax.experimental.pallas import tpu as pltpu
from jax.experimental.pallas import tpu_sc as plsc

N = 10000
NP = 10240
E = 320000
D = 128
NC = 2
NS = 16
NW = NC * NS
EPW = E // NW
CH = 80
NCHUNK = EPW // CH
RPT = NP // NS

_MESH = plsc.VectorSubcoreMesh(
    core_axis_name="c", subcore_axis_name="s", num_cores=NC, num_subcores=NS)


def _wid():
  return lax.axis_index("c") * NS + lax.axis_index("s")


_DEGW = 4


@functools.partial(
    pl.kernel,
    out_type=jax.ShapeDtypeStruct((NC, NP, 16), jnp.float32),
    mesh=_MESH,
    scratch_types=[
        pltpu.VMEM((EPW,), jnp.int32),
        pltpu.VMEM((CH, 16), jnp.float32),
        pltpu.VMEM((128, 16), jnp.float32),
        pltpu.VMEM_SHARED((NP, 16), jnp.float32),
        pltpu.SemaphoreType.DMA,
    ],
)
def _deg_kernel(edst_hbm, out_hbm, dst_v, ones_v, zero_v, acc_sh, sem):
  c = lax.axis_index("c")
  s = lax.axis_index("s")
  wid = c * NS + s
  pltpu.sync_copy(edst_hbm.at[pl.ds(wid * EPW, EPW)], dst_v)
  one = jnp.ones((16,), jnp.float32)
  zero = jnp.zeros((16,), jnp.float32)
  for i in range(CH):
    ones_v[i] = one
  for i in range(128):
    zero_v[i] = zero
  for k in range(RPT // 128):
    pltpu.sync_copy(zero_v, acc_sh.at[pl.ds(s * RPT + k * 128, 128)])
  plsc.subcore_barrier()

  for k in range(_DEGW):
    pltpu.async_copy(ones_v, acc_sh.at[dst_v.at[pl.ds(k * CH, CH)]], sem,
                     add=True)

  def body(k, carry):
    pltpu.make_async_copy(
        ones_v, acc_sh.at[dst_v.at[pl.ds(0, CH)]], sem).wait()
    off = pl.multiple_of(k * CH, 8)
    pltpu.async_copy(ones_v, acc_sh.at[dst_v.at[pl.ds(off, CH)]], sem,
                     add=True)
    return carry

  lax.fori_loop(_DEGW, NCHUNK, body, 0)
  for _ in range(_DEGW):
    pltpu.make_async_copy(
        ones_v, acc_sh.at[dst_v.at[pl.ds(0, CH)]], sem).wait()
  plsc.subcore_barrier()
  pltpu.sync_copy(acc_sh.at[pl.ds(s * RPT, RPT)],
                  out_hbm.at[c].at[pl.ds(s * RPT, RPT)])


@functools.partial(
    pl.kernel,
    out_type=jax.ShapeDtypeStruct((NC, NP, D), jnp.float32),
    mesh=_MESH,
    scratch_types=[
        pltpu.VMEM((EPW,), jnp.int32),
        pltpu.VMEM((EPW,), jnp.int32),
        pltpu.VMEM((CH, D), jnp.float32),
        pltpu.VMEM((CH, D), jnp.float32),
        pltpu.VMEM((32, D), jnp.float32),
        pltpu.VMEM_SHARED((NP, D), jnp.float32),
        pltpu.SemaphoreType.DMA,
        pltpu.SemaphoreType.DMA,
        pltpu.SemaphoreType.DMA,
        pltpu.SemaphoreType.DMA,
    ],
)
def _agg_kernel(y_hbm, esrc_hbm, edst_hbm, out_hbm,
                src_v, dst_v, rows0, rows1, zero_v, acc_sh,
                gs0, gs1, ss0, ss1):
  c = lax.axis_index("c")
  s = lax.axis_index("s")
  wid = c * NS + s
  pltpu.sync_copy(esrc_hbm.at[pl.ds(wid * EPW, EPW)], src_v)
  pltpu.sync_copy(edst_hbm.at[pl.ds(wid * EPW, EPW)], dst_v)
  zero = jnp.zeros((16,), jnp.float32)
  for i in range(32):
    for j in range(D // 16):
      zero_v[i, pl.ds(j * 16, 16)] = zero
  for k in range(RPT // 32):
    pltpu.sync_copy(zero_v, acc_sh.at[pl.ds(s * RPT + k * 32, 32)])
  plsc.subcore_barrier()

  def gat(k, buf, sem):
    off = pl.multiple_of(k * CH, 8)
    pltpu.async_copy(y_hbm.at[src_v.at[pl.ds(off, CH)]], buf, sem)

  def gat_wait(buf, sem):
    pltpu.make_async_copy(
        y_hbm.at[src_v.at[pl.ds(0, CH)]], buf, sem).wait()

  def sca(k, buf, sem):
    off = pl.multiple_of(k * CH, 8)
    pltpu.async_copy(buf, acc_sh.at[dst_v.at[pl.ds(off, CH)]], sem, add=True)

  def sca_wait(buf, sem):
    pltpu.make_async_copy(
        buf, acc_sh.at[dst_v.at[pl.ds(0, CH)]], sem).wait()

  gat(0, rows0, gs0)

  def pair(kk, carry):
    k0 = kk * 2
    gat(k0 + 1, rows1, gs1)
    gat_wait(rows0, gs0)
    sca(k0, rows0, ss0)
    sca_wait(rows0, ss0)
    gat(k0 + 2, rows0, gs0)
    gat_wait(rows1, gs1)
    sca(k0 + 1, rows1, ss1)
    sca_wait(rows1, ss1)
    return carry

  lax.fori_loop(0, (NCHUNK - 1) // 2, pair, 0)
  gat_wait(rows0, gs0)
  sca(NCHUNK - 1, rows0, ss0)
  sca_wait(rows0, ss0)

  plsc.subcore_barrier()
  pltpu.sync_copy(acc_sh.at[pl.ds(s * RPT, RPT)],
                  out_hbm.at[c].at[pl.ds(s * RPT, RPT)])


_EC = 2000


@functools.partial(
    pl.kernel,
    out_type=jax.ShapeDtypeStruct((E * 4,), jnp.float32),
    mesh=_MESH,
    scratch_types=[
        pltpu.VMEM((N * 8,), jnp.float32),
        pltpu.VMEM((_EC,), jnp.int32),
        pltpu.VMEM((_EC,), jnp.int32),
        pltpu.VMEM((_EC * 4,), jnp.float32),
    ],
    compiler_params=pltpu.CompilerParams(needs_layout_passes=False),
)
def _edge_out_kernel(ab_hbm, src_hbm, dst_hbm, out_hbm,
                     ab_v, src_v, dst_v, out_v):
  wid = _wid()
  pltpu.sync_copy(ab_hbm, ab_v)
  lanes = lax.iota(jnp.int32, 16)

  def chunk(kc, carry):
    base = pl.multiple_of(wid * EPW + kc * _EC, 8)
    pltpu.sync_copy(src_hbm.at[pl.ds(base, _EC)], src_v)
    pltpu.sync_copy(dst_hbm.at[pl.ds(base, _EC)], dst_v)

    def group(g, carry2):
      off = pl.multiple_of(g * 16, 16)
      sv = src_v[pl.ds(off, 16)] * 8
      dv = dst_v[pl.ds(off, 16)] * 8
      erow = (g * 16 + lanes) * 4
      for j in range(4):
        a = plsc.load_gather(ab_v, [sv + j])
        b = plsc.load_gather(ab_v, [dv + (j + 4)])
        plsc.store_scatter(out_v, [erow + j], a + b)
      return carry2

    lax.fori_loop(0, _EC // 16, group, 0)
    pltpu.sync_copy(out_v, out_hbm.at[pl.ds(base * 4, _EC * 4)])
    return carry

  lax.fori_loop(0, EPW // _EC, chunk, 0)


_BLK = 1000
_GRID = N // _BLK


def _dinv_block(degp):
  deg = degp[0, :, 0] + degp[1, :, 0] + 1.0
  return lax.rsqrt(deg)[:, None]


def _tc1_body(x_ref, w_ref, degp_ref, y_ref):
  dinv = _dinv_block(degp_ref[...])
  y_ref[...] = jnp.dot(x_ref[...], w_ref[...],
                       preferred_element_type=jnp.float32) * dinv


def _tc2_body(accp_ref, y_ref, degp_ref, b_ref, w_ref, y2_ref):
  dinv = _dinv_block(degp_ref[...])
  a = accp_ref[...]
  h = jnp.maximum(dinv * (a[0] + a[1] + y_ref[...]) + b_ref[...], 0.0)
  y2_ref[...] = jnp.dot(h, w_ref[...],
                        preferred_element_type=jnp.float32) * dinv


def _tc3_body(accp_ref, y_ref, degp_ref, b_ref,
              m1_ref, m2_ref, m3_ref, mb1_ref, mb2_ref, mb3_ref, ab_ref):
  dinv = _dinv_block(degp_ref[...])
  a = accp_ref[...]
  h2 = jnp.maximum(dinv * (a[0] + a[1] + y_ref[...]) + b_ref[...], 0.0)
  m23 = jnp.dot(m2_ref[...], m3_ref[...], preferred_element_type=jnp.float32)
  mc = jnp.dot(m1_ref[...], m23, preferred_element_type=jnp.float32)
  bc = (jnp.dot(mb1_ref[...], m23, preferred_element_type=jnp.float32)
        + jnp.dot(mb2_ref[...], m3_ref[...],
                  preferred_element_type=jnp.float32)
        + mb3_ref[...])
  mcc = jnp.concatenate([mc[:D], mc[D:]], axis=1)
  ab = jnp.dot(h2, mcc, preferred_element_type=jnp.float32)
  ab_ref[...] = ab + jnp.concatenate(
      [jnp.zeros((1, 4), jnp.float32), bc], axis=1)


def _row_spec(w=D):
  return pl.BlockSpec((_BLK, w), lambda i: (i, 0))


_DEGP_SPEC = pl.BlockSpec((NC, _BLK, 16), lambda i: (0, i, 0))
_ACCP_SPEC = pl.BlockSpec((NC, _BLK, D), lambda i: (0, i, 0))


def _full(*shape):
  return pl.BlockSpec(shape, lambda i: tuple(0 for _ in shape))


def kernel(x, edge_index, W1, b1, W2, b2, M1, mb1, M2, mb2, M3, mb3):
  src = edge_index[0].astype(jnp.int32)
  dst = edge_index[1].astype(jnp.int32)
  degp = _deg_kernel(dst)

  y1 = pl.pallas_call(
      _tc1_body,
      grid=(_GRID,),
      in_specs=[_row_spec(), _full(D, D), _DEGP_SPEC],
      out_specs=_row_spec(),
      out_shape=jax.ShapeDtypeStruct((N, D), jnp.float32),
  )(x, W1, degp)

  acc1 = _agg_kernel(y1, src, dst)

  y2 = pl.pallas_call(
      _tc2_body,
      grid=(_GRID,),
      in_specs=[_ACCP_SPEC, _row_spec(), _DEGP_SPEC, _full(1, D),
                _full(D, D)],
      out_specs=_row_spec(),
      out_shape=jax.ShapeDtypeStruct((N, D), jnp.float32),
  )(acc1, y1, degp, b1.reshape(1, D), W2)

  acc2 = _agg_kernel(y2, src, dst)

  ab = pl.pallas_call(
      _tc3_body,
      grid=(_GRID,),
      in_specs=[_ACCP_SPEC, _row_spec(), _DEGP_SPEC, _full(1, D),
                _full(2 * D, 64), _full(64, 32), _full(32, 4),
                _full(1, 64), _full(1, 32), _full(1, 4)],
      out_specs=_row_spec(8),
      out_shape=jax.ShapeDtypeStruct((N, 8), jnp.float32),
  )(acc2, y2, degp, b2.reshape(1, D), M1, M2, M3,
    mb1.reshape(1, 64), mb2.reshape(1, 32), mb3.reshape(1, 4))

  return _edge_out_kernel(ab.reshape(N * 8), src, dst).reshape(E, 4)


# speedup vs baseline: 28.5552x; 1.6238x over previous
import functools

import jax
import jax.numpy as jnp
from jax import lax
from jax.experimental import pallas as pl
from jax.experimental.pallas import tpu as pltpu
from jax.experimental.pallas import tpu_sc as plsc

N = 10000
NP = 10240
E = 320000
D = 128
NC = 2
NS = 16
NW = NC * NS
EPW = E // NW
CH = 80
NCHUNK = EPW // CH
RPT = NP // NS
ACH = 104
ANCH = EPW // ACH
ATL = EPW - ANCH * ACH

_MESH = plsc.VectorSubcoreMesh(
    core_axis_name="c", subcore_axis_name="s", num_cores=NC, num_subcores=NS)


def _wid():
  return lax.axis_index("c") * NS + lax.axis_index("s")


_DEGW = 4


@functools.partial(
    pl.kernel,
    out_type=jax.ShapeDtypeStruct((NC, NP, 16), jnp.float32),
    mesh=_MESH,
    scratch_types=[
        pltpu.VMEM((EPW,), jnp.int32),
        pltpu.VMEM((CH, 16), jnp.float32),
        pltpu.VMEM((128, 16), jnp.float32),
        pltpu.VMEM_SHARED((NP, 16), jnp.float32),
        pltpu.SemaphoreType.DMA,
    ],
)
def _deg_kernel(edst_hbm, out_hbm, dst_v, ones_v, zero_v, acc_sh, sem):
  c = lax.axis_index("c")
  s = lax.axis_index("s")
  wid = c * NS + s
  pltpu.sync_copy(edst_hbm.at[pl.ds(wid * EPW, EPW)], dst_v)
  one = jnp.ones((16,), jnp.float32)
  zero = jnp.zeros((16,), jnp.float32)
  for i in range(CH):
    ones_v[i] = one
  for i in range(128):
    zero_v[i] = zero
  for k in range(RPT // 128):
    pltpu.sync_copy(zero_v, acc_sh.at[pl.ds(s * RPT + k * 128, 128)])
  plsc.subcore_barrier()

  for k in range(_DEGW):
    pltpu.async_copy(ones_v, acc_sh.at[dst_v.at[pl.ds(k * CH, CH)]], sem,
                     add=True)

  def body(k, carry):
    pltpu.make_async_copy(
        ones_v, acc_sh.at[dst_v.at[pl.ds(0, CH)]], sem).wait()
    off = pl.multiple_of(k * CH, 8)
    pltpu.async_copy(ones_v, acc_sh.at[dst_v.at[pl.ds(off, CH)]], sem,
                     add=True)
    return carry

  lax.fori_loop(_DEGW, NCHUNK, body, 0)
  for _ in range(_DEGW):
    pltpu.make_async_copy(
        ones_v, acc_sh.at[dst_v.at[pl.ds(0, CH)]], sem).wait()
  plsc.subcore_barrier()
  pltpu.sync_copy(acc_sh.at[pl.ds(s * RPT, RPT)],
                  out_hbm.at[c].at[pl.ds(s * RPT, RPT)])


@functools.partial(
    pl.kernel,
    out_type=jax.ShapeDtypeStruct((NC, NP, D), jnp.float32),
    mesh=_MESH,
    scratch_types=[
        pltpu.VMEM((EPW,), jnp.int32),
        pltpu.VMEM((EPW,), jnp.int32),
        pltpu.VMEM((ACH, D), jnp.float32),
        pltpu.VMEM((ACH, D), jnp.float32),
        pltpu.VMEM((16, D), jnp.float32),
        pltpu.VMEM_SHARED((NP, D), jnp.float32),
        pltpu.SemaphoreType.DMA,
        pltpu.SemaphoreType.DMA,
        pltpu.SemaphoreType.DMA,
        pltpu.SemaphoreType.DMA,
    ],
)
def _agg_kernel(y_hbm, esrc_hbm, edst_hbm, out_hbm,
                src_v, dst_v, rows0, rows1, zero_v, acc_sh,
                gs0, gs1, ss0, ss1):
  c = lax.axis_index("c")
  s = lax.axis_index("s")
  wid = c * NS + s
  pltpu.sync_copy(esrc_hbm.at[pl.ds(wid * EPW, EPW)], src_v)
  pltpu.sync_copy(edst_hbm.at[pl.ds(wid * EPW, EPW)], dst_v)
  zero = jnp.zeros((16,), jnp.float32)
  for i in range(16):
    for j in range(D // 16):
      zero_v[i, pl.ds(j * 16, 16)] = zero
  for k in range(RPT // 16):
    pltpu.sync_copy(zero_v, acc_sh.at[pl.ds(s * RPT + k * 16, 16)])
  plsc.subcore_barrier()

  def gat(k, buf, sem):
    off = pl.multiple_of(k * ACH, 8)
    pltpu.async_copy(y_hbm.at[src_v.at[pl.ds(off, ACH)]], buf, sem)

  def gat_wait(buf, sem):
    pltpu.make_async_copy(
        y_hbm.at[src_v.at[pl.ds(0, ACH)]], buf, sem).wait()

  def sca(k, buf, sem):
    off = pl.multiple_of(k * ACH, 8)
    pltpu.async_copy(buf, acc_sh.at[dst_v.at[pl.ds(off, ACH)]], sem, add=True)

  def sca_wait(buf, sem):
    pltpu.make_async_copy(
        buf, acc_sh.at[dst_v.at[pl.ds(0, ACH)]], sem).wait()

  gat(0, rows0, gs0)

  def pair(kk, carry):
    k0 = kk * 2
    gat(k0 + 1, rows1, gs1)
    gat_wait(rows0, gs0)
    sca(k0, rows0, ss0)
    sca_wait(rows0, ss0)

    @pl.when(k0 + 2 < ANCH)
    def _():
      gat(k0 + 2, rows0, gs0)

    gat_wait(rows1, gs1)
    sca(k0 + 1, rows1, ss1)
    sca_wait(rows1, ss1)
    return carry

  lax.fori_loop(0, ANCH // 2, pair, 0)
  toff = pl.multiple_of(ANCH * ACH, 8)
  tb0 = rows0.at[pl.ds(0, ATL)]
  pltpu.async_copy(y_hbm.at[src_v.at[pl.ds(toff, ATL)]], tb0, gs0)
  pltpu.make_async_copy(y_hbm.at[src_v.at[pl.ds(0, ATL)]], tb0, gs0).wait()
  pltpu.async_copy(tb0, acc_sh.at[dst_v.at[pl.ds(toff, ATL)]], ss0, add=True)
  pltpu.make_async_copy(tb0, acc_sh.at[dst_v.at[pl.ds(0, ATL)]], ss0).wait()

  plsc.subcore_barrier()
  pltpu.sync_copy(acc_sh.at[pl.ds(s * RPT, RPT)],
                  out_hbm.at[c].at[pl.ds(s * RPT, RPT)])


_EC = 2000


@functools.partial(
    pl.kernel,
    out_type=[jax.ShapeDtypeStruct((E,), jnp.float32) for _ in range(4)],
    mesh=_MESH,
    scratch_types=[
        pltpu.VMEM((N * 8,), jnp.float32),
        pltpu.VMEM((EPW,), jnp.int32),
        pltpu.VMEM((EPW,), jnp.int32),
    ] + [pltpu.VMEM((_EC,), jnp.float32) for _ in range(8)] + [
        pltpu.SemaphoreType.DMA,
        pltpu.SemaphoreType.DMA,
    ],
    compiler_params=pltpu.CompilerParams(needs_layout_passes=False),
)
def _edge_out_kernel(ab_hbm, src_hbm, dst_hbm, o0_hbm, o1_hbm, o2_hbm,
                     o3_hbm, ab_v, src_v, dst_v, b00, b01, b02, b03,
                     b10, b11, b12, b13, sem0, sem1):
  wid = _wid()
  base0 = pl.multiple_of(wid * EPW, 8)
  pltpu.sync_copy(src_hbm.at[pl.ds(base0, EPW)], src_v)
  pltpu.sync_copy(dst_hbm.at[pl.ds(base0, EPW)], dst_v)
  pltpu.sync_copy(ab_hbm, ab_v)
  lanes = lax.iota(jnp.int32, 16)
  outs_hbm = (o0_hbm, o1_hbm, o2_hbm, o3_hbm)
  slot_bufs = ((b00, b01, b02, b03), (b10, b11, b12, b13))

  def do_chunk(kc, slot, sem, wait_first):
    bufs = slot_bufs[slot]
    def group(g, carry2):
      off = pl.multiple_of(kc * _EC + g * 16, 16)
      sv = src_v[pl.ds(off, 16)] * 8
      dv = dst_v[pl.ds(off, 16)] * 8
      erow = g * 16 + lanes
      for j in range(4):
        a = plsc.load_gather(ab_v, [sv + j])
        b = plsc.load_gather(ab_v, [dv + (j + 4)])
        plsc.store_scatter(bufs[j], [erow], a + b)
      return carry2

    @pl.when(wait_first)
    def _():
      for j in range(4):
        pltpu.make_async_copy(
            bufs[j], outs_hbm[j].at[pl.ds(0, _EC)], sem).wait()

    lax.fori_loop(0, _EC // 16, group, 0)
    base = pl.multiple_of(wid * EPW + kc * _EC, 8)
    for j in range(4):
      pltpu.async_copy(bufs[j], outs_hbm[j].at[pl.ds(base, _EC)], sem)

  def chunk(kc, carry):
    do_chunk(kc * 2, 0, sem0, kc > 0)
    do_chunk(kc * 2 + 1, 1, sem1, kc > 0)
    return carry

  lax.fori_loop(0, EPW // _EC // 2, chunk, 0)
  do_chunk(EPW // _EC - 1, 0, sem0, True)
  for j in range(4):
    pltpu.make_async_copy(
        slot_bufs[0][j], outs_hbm[j].at[pl.ds(0, _EC)], sem0).wait()
    pltpu.make_async_copy(
        slot_bufs[1][j], outs_hbm[j].at[pl.ds(0, _EC)], sem1).wait()


_BLK = 1000
_GRID = N // _BLK


def _dinv_block(degp):
  deg = degp[0, :, 0] + degp[1, :, 0] + 1.0
  return lax.rsqrt(deg)[:, None]


def _tca_body(x_ref, w_ref, u_ref):
  u_ref[...] = jnp.dot(x_ref[...], w_ref[...],
                       preferred_element_type=jnp.float32)


def _tcb_body(u_ref, degp_ref, y_ref):
  y_ref[...] = u_ref[...] * _dinv_block(degp_ref[...])


def _tc2_body(accp_ref, y_ref, degp_ref, b_ref, w_ref, y2_ref):
  dinv = _dinv_block(degp_ref[...])
  a = accp_ref[...]
  h = jnp.maximum(dinv * (a[0] + a[1] + y_ref[...]) + b_ref[...], 0.0)
  y2_ref[...] = jnp.dot(h, w_ref[...],
                        preferred_element_type=jnp.float32) * dinv


def _tc3_body(accp_ref, y_ref, degp_ref, b_ref,
              m1_ref, m2_ref, m3_ref, mb1_ref, mb2_ref, mb3_ref, ab_ref):
  dinv = _dinv_block(degp_ref[...])
  a = accp_ref[...]
  h2 = jnp.maximum(dinv * (a[0] + a[1] + y_ref[...]) + b_ref[...], 0.0)
  m23 = jnp.dot(m2_ref[...], m3_ref[...], preferred_element_type=jnp.float32)
  mc = jnp.dot(m1_ref[...], m23, preferred_element_type=jnp.float32)
  bc = (jnp.dot(mb1_ref[...], m23, preferred_element_type=jnp.float32)
        + jnp.dot(mb2_ref[...], m3_ref[...],
                  preferred_element_type=jnp.float32)
        + mb3_ref[...])
  mcc = jnp.concatenate([mc[:D], mc[D:]], axis=1)
  ab = jnp.dot(h2, mcc, preferred_element_type=jnp.float32)
  ab_ref[...] = ab + jnp.concatenate(
      [jnp.zeros((1, 4), jnp.float32), bc], axis=1)


def _row_spec(w=D):
  return pl.BlockSpec((_BLK, w), lambda i: (i, 0))


_DEGP_SPEC = pl.BlockSpec((NC, _BLK, 16), lambda i: (0, i, 0))
_ACCP_SPEC = pl.BlockSpec((NC, _BLK, D), lambda i: (0, i, 0))


def _full(*shape):
  return pl.BlockSpec(shape, lambda i: tuple(0 for _ in shape))


def kernel(x, edge_index, W1, b1, W2, b2, M1, mb1, M2, mb2, M3, mb3):
  src = edge_index[0].astype(jnp.int32)
  dst = edge_index[1].astype(jnp.int32)
  degp = _deg_kernel(dst)

  u1 = pl.pallas_call(
      _tca_body,
      grid=(_GRID,),
      in_specs=[_row_spec(), _full(D, D)],
      out_specs=_row_spec(),
      out_shape=jax.ShapeDtypeStruct((N, D), jnp.float32),
  )(x, W1)

  y1 = pl.pallas_call(
      _tcb_body,
      grid=(_GRID,),
      in_specs=[_row_spec(), _DEGP_SPEC],
      out_specs=_row_spec(),
      out_shape=jax.ShapeDtypeStruct((N, D), jnp.float32),
  )(u1, degp)

  acc1 = _agg_kernel(y1, src, dst)

  y2 = pl.pallas_call(
      _tc2_body,
      grid=(_GRID,),
      in_specs=[_ACCP_SPEC, _row_spec(), _DEGP_SPEC, _full(1, D),
                _full(D, D)],
      out_specs=_row_spec(),
      out_shape=jax.ShapeDtypeStruct((N, D), jnp.float32),
  )(acc1, y1, degp, b1.reshape(1, D), W2)

  acc2 = _agg_kernel(y2, src, dst)

  ab = pl.pallas_call(
      _tc3_body,
      grid=(_GRID,),
      in_specs=[_ACCP_SPEC, _row_spec(), _DEGP_SPEC, _full(1, D),
                _full(2 * D, 64), _full(64, 32), _full(32, 4),
                _full(1, 64), _full(1, 32), _full(1, 4)],
      out_specs=_row_spec(8),
      out_shape=jax.ShapeDtypeStruct((N, 8), jnp.float32),
  )(acc2, y2, degp, b2.reshape(1, D), M1, M2, M3,
    mb1.reshape(1, 64), mb2.reshape(1, 32), mb3.reshape(1, 4))

  cols = _edge_out_kernel(ab.reshape(N * 8), src, dst)
  return jnp.stack(cols, axis=1)
